# baseline (speedup 1.0000x reference)
import dataclasses
import functools

import jax
import jax.numpy as jnp
from jax import lax
from jax.experimental import pallas as pl
from jax.experimental.pallas import tpu as pltpu
from jax.experimental.pallas import tpu_sc as plsc

_NUM_CORES = 2
_NUM_SUBCORES = 16
_NUM_WORKERS = _NUM_CORES * _NUM_SUBCORES
_ROWS = 256
_PAIRS = _ROWS // 2
_LANES = 16


def _compiler_params():
    cp = pltpu.CompilerParams(use_tc_tiling_on_sc=True)
    if "needs_layout_passes" in pltpu.CompilerParams.__dataclass_fields__:
        cp = dataclasses.replace(cp, needs_layout_passes=False)
    return cp


def _gather_rows(idx_flat, table2, n):
    n_per_w = n // _NUM_WORKERS
    n_chunks = n_per_w // _ROWS
    mesh = plsc.VectorSubcoreMesh(core_axis_name="c", subcore_axis_name="s")

    @functools.partial(
        pl.kernel,
        mesh=mesh,
        out_type=jax.ShapeDtypeStruct((n // 2, 128), jnp.float32),
        scratch_types=[
            pltpu.VMEM((_ROWS,), jnp.int32),
            pltpu.VMEM((_ROWS,), jnp.int32),
            pltpu.VMEM((_ROWS, 128), jnp.float32),
            pltpu.VMEM((_PAIRS, 128), jnp.float32),
            pltpu.SemaphoreType.DMA,
        ],
        compiler_params=_compiler_params(),
    )
    def gather_kernel(table_hbm, idx_hbm, out_hbm, idx_v, pidx_v, buf, stag, sem):
        wid = lax.axis_index("s") * _NUM_CORES + lax.axis_index("c")
        base = wid * n_per_w

        @pl.loop(0, n_chunks)
        def _(i):
            off = base + i * _ROWS
            pltpu.sync_copy(idx_hbm.at[pl.ds(off, _ROWS)], idx_v)

            @pl.loop(0, _ROWS // _LANES)
            def _(v):
                sl = pl.ds(v * _LANES, _LANES)
                pidx_v[sl] = lax.shift_right_logical(idx_v[sl], 1)

            pltpu.async_copy(table_hbm.at[pidx_v], buf, sem).wait()

            iota = lax.iota(jnp.int32, _LANES)
            for h in range(2):
                @pl.loop(0, _PAIRS // _LANES)
                def _(pb, h=h):
                    m_vec = pb * _LANES + iota
                    row_vec = 2 * m_vec + h
                    par = lax.bitwise_and(
                        plsc.load_gather(idx_v, [row_vec]), 1)
                    col_base = par * 64

                    @pl.loop(0, 64)
                    def _(c):
                        vals = plsc.load_gather(buf, [row_vec, col_base + c])
                        plsc.store_scatter(
                            stag, [m_vec, jnp.full((_LANES,), h * 64, jnp.int32) + c],
                            vals)

            pout = pl.multiple_of(off // 2, 8)
            pltpu.sync_copy(stag, out_hbm.at[pl.ds(pout, _PAIRS)])

    return gather_kernel(table2, idx_flat)


def kernel(input_batch, table):
    b, l = input_batch.shape
    v, d = table.shape
    n = b * l
    idx_flat = input_batch.reshape(n)
    table2 = table.reshape(v // 2, 2 * d)
    out2 = _gather_rows(idx_flat, table2, n)
    embedded = out2.reshape(b, l, d)
    sequence_lengths = jnp.full((b,), float(l), dtype=jnp.float32)
    return (embedded, sequence_lengths)


# speedup vs baseline: 1.4416x; 1.4416x over previous
import dataclasses
import functools

import jax
import jax.numpy as jnp
from jax import lax
from jax.experimental import pallas as pl
from jax.experimental.pallas import tpu as pltpu
from jax.experimental.pallas import tpu_sc as plsc

_NUM_CORES = 2
_NUM_SUBCORES = 16
_NUM_WORKERS = _NUM_CORES * _NUM_SUBCORES
_ROWS = 320
_PAIRS = _ROWS // 2
_LANES = 16


def _compiler_params():
    cp = pltpu.CompilerParams(use_tc_tiling_on_sc=True)
    if "needs_layout_passes" in pltpu.CompilerParams.__dataclass_fields__:
        cp = dataclasses.replace(cp, needs_layout_passes=False)
    return cp


def _gather_rows(idx_flat, table2, n):
    n_per_w = n // _NUM_WORKERS
    n_chunks = n_per_w // _ROWS
    mesh = plsc.VectorSubcoreMesh(core_axis_name="c", subcore_axis_name="s")

    @functools.partial(
        pl.kernel,
        mesh=mesh,
        out_type=jax.ShapeDtypeStruct((n // 2, 128), jnp.float32),
        scratch_types=[
            pltpu.VMEM((2 * _ROWS,), jnp.int32),
            pltpu.VMEM((2 * _ROWS, 128), jnp.float32),
            pltpu.VMEM((2 * _PAIRS, 128), jnp.float32),
            pltpu.VMEM((2 * _ROWS,), jnp.int32),
            pltpu.SemaphoreType.DMA,
            pltpu.SemaphoreType.DMA,
            pltpu.SemaphoreType.DMA,
        ],
        compiler_params=_compiler_params(),
    )
    def gather_kernel(table_hbm, idx_hbm, out_hbm, pidx_v, buf, stag,
                      par_v, gsem0, gsem1, ssem):
        wid = lax.axis_index("s") * _NUM_CORES + lax.axis_index("c")
        base = wid * n_per_w
        gsems = (gsem0, gsem1)

        def start_gather(i, slot):
            off = base + i * _ROWS
            spidx = pidx_v.at[pl.ds(slot * _ROWS, _ROWS)]
            spar = par_v.at[pl.ds(slot * _ROWS, _ROWS)]
            pltpu.sync_copy(idx_hbm.at[pl.ds(off, _ROWS)], spidx)

            @pl.loop(0, _ROWS // _LANES)
            def _(v):
                sl = pl.ds(v * _LANES, _LANES)
                raw = spidx[sl]
                spar[sl] = lax.bitwise_and(raw, 1) * 64
                spidx[sl] = lax.shift_right_logical(raw, 1)

            return pltpu.async_copy(
                table_hbm.at[spidx],
                buf.at[pl.ds(slot * _ROWS, _ROWS)], gsems[slot])

        def extract_and_store(i, slot):
            bbuf = buf.at[pl.ds(slot * _ROWS, _ROWS)]
            sstag = stag.at[pl.ds(slot * _PAIRS, _PAIRS)]

            sbase = slot * _ROWS

            @pl.loop(0, _PAIRS)
            def _(m):
                r0 = jnp.full((_LANES,), sbase + 2 * m, jnp.int32)
                p0 = plsc.load_gather(par_v, [r0]) > 0
                p1 = plsc.load_gather(par_v, [r0 + 1]) > 0
                for k in range(4):
                    lo0 = bbuf[2 * m, pl.ds(16 * k, 16)]
                    hi0 = bbuf[2 * m, pl.ds(64 + 16 * k, 16)]
                    sstag[m, pl.ds(16 * k, 16)] = jnp.where(p0, hi0, lo0)
                    lo1 = bbuf[2 * m + 1, pl.ds(16 * k, 16)]
                    hi1 = bbuf[2 * m + 1, pl.ds(64 + 16 * k, 16)]
                    sstag[m, pl.ds(64 + 16 * k, 16)] = jnp.where(p1, hi1, lo1)

            off = base + i * _ROWS
            pout = pl.multiple_of(off // 2, 8)
            pltpu.async_copy(sstag, out_hbm.at[pl.ds(pout, _PAIRS)],
                             ssem).wait()

        handles = {0: start_gather(0, 0)}
        for i in range(n_chunks):
            if i + 1 < n_chunks:
                handles[i + 1] = start_gather(i + 1, (i + 1) % 2)
            handles.pop(i).wait()
            extract_and_store(i, i % 2)

    return gather_kernel(table2, idx_flat)


def kernel(input_batch, table):
    b, l = input_batch.shape
    v, d = table.shape
    n = b * l
    idx_flat = input_batch.reshape(n)
    table2 = table.reshape(v // 2, 2 * d)
    out2 = _gather_rows(idx_flat, table2, n)
    embedded = out2.reshape(b, l, d)
    sequence_lengths = jnp.full((b,), float(l), dtype=jnp.float32)
    return (embedded, sequence_lengths)
